# baseline (speedup 1.0000x reference)
import functools

import jax
import jax.numpy as jnp
from jax import lax
from jax.experimental import pallas as pl
from jax.experimental.pallas import tpu as pltpu
from jax.experimental.pallas import tpu_sc as plsc

NW = 32
SUB = 128


def _make_gather(B, V, D, C):
  n_sub = C // SUB
  b_per_w = B // NW
  n_chunks = b_per_w // C
  mesh = plsc.VectorSubcoreMesh(core_axis_name="c", subcore_axis_name="s")

  @functools.partial(
      pl.kernel,
      mesh=mesh,
      out_type=jax.ShapeDtypeStruct((B, D), jnp.float32),
      compiler_params=pltpu.CompilerParams(use_tc_tiling_on_sc=False),
      scratch_types=[
          pltpu.VMEM((n_sub, SUB), jnp.int32),
          pltpu.VMEM((C, D), jnp.float32),
          pltpu.SemaphoreType.DMA,
      ],
  )
  def body(idx_hbm, table_hbm, out_hbm, idx_v, rows_v, sem):
    cid = lax.axis_index("c")
    sid = lax.axis_index("s")
    wid = sid * 2 + cid
    base_row = wid * (b_per_w // SUB)

    def chunk(i, carry):
      row0 = base_row + i * n_sub
      pltpu.sync_copy(idx_hbm.at[pl.ds(row0, n_sub)], idx_v)
      cps = []
      for j in range(n_sub):
        cp = pltpu.make_async_copy(
            table_hbm.at[idx_v.at[j]],
            rows_v.at[pl.ds(j * SUB, SUB)],
            sem,
        )
        cp.start()
        cps.append(cp)
      for cp in cps:
        cp.wait()
      pltpu.sync_copy(rows_v, out_hbm.at[pl.ds(row0 * SUB, C)])
      return carry

    lax.fori_loop(0, n_chunks, chunk, 0)

  return body


def kernel(inputs, weight):
  B0, S = inputs.shape
  V, D = weight.shape
  B = B0 * S
  idx = inputs.reshape(B // SUB, SUB).astype(jnp.int32)
  out = _make_gather(B, V, D, C=512)(idx, weight)
  return out.reshape(B0, S, D)


# speedup vs baseline: 1.0463x; 1.0463x over previous
import functools

import jax
import jax.numpy as jnp
from jax import lax
from jax.experimental import pallas as pl
from jax.experimental.pallas import tpu as pltpu
from jax.experimental.pallas import tpu_sc as plsc

NW = 32
SUB = 128


def _make_gather(B, V, D, C):
  n_sub = C // SUB
  b_per_w = B // NW
  n_chunks = b_per_w // C
  idx_rows = b_per_w // SUB
  assert n_chunks % 2 == 0
  mesh = plsc.VectorSubcoreMesh(core_axis_name="c", subcore_axis_name="s")

  @functools.partial(
      pl.kernel,
      mesh=mesh,
      out_type=jax.ShapeDtypeStruct((B, D), jnp.float32),
      compiler_params=pltpu.CompilerParams(use_tc_tiling_on_sc=False),
      scratch_types=[
          pltpu.VMEM((idx_rows, SUB), jnp.int32),
          pltpu.VMEM((2, C, D), jnp.float32),
          pltpu.SemaphoreType.DMA,
          pltpu.SemaphoreType.DMA,
          pltpu.SemaphoreType.DMA,
          pltpu.SemaphoreType.DMA,
      ],
  )
  def body(idx_hbm, table_hbm, out_hbm, idx_v, rows_v, g0, g1, o0, o1):
    cid = lax.axis_index("c")
    sid = lax.axis_index("s")
    wid = sid * 2 + cid
    base_row = wid * idx_rows
    gsem = (g0, g1)
    osem = (o0, o1)

    pltpu.sync_copy(idx_hbm.at[pl.ds(base_row, idx_rows)], idx_v)

    def gathers(i, b):
      rb = rows_v.at[b]
      return [
          pltpu.make_async_copy(
              table_hbm.at[idx_v.at[i * n_sub + j]],
              rb.at[pl.ds(j * SUB, SUB)],
              gsem[b],
          )
          for j in range(n_sub)
      ]

    def store(i, b):
      return pltpu.make_async_copy(
          rows_v.at[b],
          out_hbm.at[pl.ds((base_row + i * n_sub) * SUB, C)],
          osem[b],
      )

    for cp in gathers(0, 0):
      cp.start()

    def half(i_cur, b):
      nb = 1 - b

      @pl.when(i_cur + 1 < n_chunks)
      def _():
        @pl.when(i_cur >= 1)
        def _():
          store(i_cur - 1, nb).wait()
        for cp in gathers(i_cur + 1, nb):
          cp.start()

      for cp in gathers(i_cur, b):
        cp.wait()
      store(i_cur, b).start()

    def pair(k, carry):
      half(2 * k, 0)
      half(2 * k + 1, 1)
      return carry

    lax.fori_loop(0, n_chunks // 2, pair, 0)
    store(n_chunks - 2, 0).wait()
    store(n_chunks - 1, 1).wait()

  return body


def kernel(inputs, weight):
  B0, S = inputs.shape
  V, D = weight.shape
  B = B0 * S
  idx = inputs.reshape(B // SUB, SUB).astype(jnp.int32)
  out = _make_gather(B, V, D, C=512)(idx, weight)
  return out.reshape(B0, S, D)
